# baseline (speedup 1.0000x reference)
import numpy as np

import jax
import jax.numpy as jnp
from jax import lax
from jax.experimental import pallas as pl
from jax.experimental.pallas import tpu as pltpu
from jax.experimental.pallas import tpu_sc as plsc

N = 10000
E = 320000
D = 128
NS = 16
NC = 2
NPAD = 10240
RPT = NPAD // NS
EC = E // NS
BLK = 128
NB = (EC + BLK - 1) // BLK
ECP = NB * BLK
CH = 32
NCH = RPT // CH

_g = np.arange(16)
_p32 = np.empty(32, np.int32)
_p32[0::2] = _g
_p32[1::2] = _g + 16
PERM = (np.arange(0, D, 32)[:, None] + _p32[None, :]).ravel()


def _rsqrt16(d):
    xi = plsc.bitcast(d, jnp.int32)
    yi = jnp.int32(0x5F3759DF) - (xi >> 1)
    y = plsc.bitcast(yi, jnp.float32)
    for _ in range(3):
        y = y * (1.5 - 0.5 * d * y * y)
    return y


def _sc_body(x_hbm, pk_hbm, xs_hbm, nd_hbm, p0_hbm, p1_hbm,
             pk, idx_src, idx_dst, rows, xrowf, xrowb, nsrc, ndbuf,
             z128, zrow, ones, acc_sh, degs_sh, degd_sh, sem, ssem):
    rel = lax.axis_index("c")
    tile = lax.axis_index("s")
    arow0 = tile * RPT

    z16 = jnp.zeros((16,), jnp.float32)
    zb = jnp.zeros((32,), jnp.bfloat16)
    for i in range(8):
        z128[pl.ds(i * 16, 16)] = z16
        ones[pl.ds(i * 16, 16)] = z16 + 1.0
        for k in range(D // 32):
            zrow[i, pl.ds(k * 32, 32)] = zb

    def zero_deg(i, _):
        pltpu.sync_copy(z128, degs_sh.at[pl.ds(rel * NPAD + arow0 + 128 * i, 128)])
        pltpu.sync_copy(z128, degd_sh.at[pl.ds(arow0 + 128 * i, 128)])
        return _
    lax.fori_loop(0, RPT // 128, zero_deg, None)

    def zero_acc(i, _):
        pltpu.sync_copy(zrow, acc_sh.at[pl.ds(arow0 + 8 * i, 8)])
        return _
    lax.fori_loop(0, RPT // 8, zero_acc, None)

    pltpu.sync_copy(pk_hbm.at[rel, tile], pk)

    def unpack_blk(j, _):
        for k in range(BLK // 16):
            s = pl.ds(k * 16, 16)
            v = pk[j, s]
            idx_src[j, s] = v >> 14
            idx_dst[j, s] = v & 16383
        return _
    lax.fori_loop(0, NB, unpack_blk, None)

    plsc.subcore_barrier()

    def deg_blk(j, _):
        pltpu.sync_copy(ones, degs_sh.at[idx_src.at[j]], add=True)
        pltpu.sync_copy(ones, degd_sh.at[idx_dst.at[j]], add=True)
        return _
    lax.fori_loop(0, NB, deg_blk, None)

    plsc.subcore_barrier()

    pltpu.sync_copy(degs_sh.at[pl.ds(rel * NPAD + arow0, RPT)], nsrc)
    pltpu.sync_copy(degd_sh.at[pl.ds(arow0, RPT)], ndbuf)

    def norm_blk(i, _):
        s = pl.ds(i * 16, 16)
        nsrc[s] = _rsqrt16(jnp.maximum(nsrc[s], 1.0))
        ndbuf[s] = _rsqrt16(jnp.maximum(ndbuf[s], 1.0))
        return _
    lax.fori_loop(0, RPT // 16, norm_blk, None)
    pltpu.sync_copy(ndbuf, nd_hbm.at[pl.ds(rel * NPAD + arow0, RPT)])

    def scale_rows(nrows, r0):
        for i in range(nrows):
            w = plsc.load_gather(nsrc, [jnp.full((16,), r0 + i, jnp.int32)])
            for k in range(D // 32):
                a = xrowf[i, pl.ds(k * 32, 16)] * w
                b = xrowf[i, pl.ds(k * 32 + 16, 16)] * w
                xrowb[i, pl.ds(k * 32, 32)] = plsc.pack(
                    a, b, format=plsc.PackFormat.INTERLEAVED)

    lim = jnp.clip(N - arow0, 0, RPT)
    nfull = lim // CH

    def stage_chunk(c, _):
        r0 = c * CH
        pltpu.sync_copy(x_hbm.at[pl.ds(arow0 + r0, CH)], xrowf)
        scale_rows(CH, r0)
        pltpu.sync_copy(
            xrowb, xs_hbm.at[pl.ds(rel * NPAD + arow0 + r0, CH)])
        return _
    lax.fori_loop(0, nfull, stage_chunk, None)

    @pl.when(nfull * CH < lim)
    def _tail():
        r0 = nfull * CH
        pltpu.sync_copy(x_hbm.at[pl.ds(arow0 + r0, CH // 2)],
                        xrowf.at[pl.ds(0, CH // 2)])
        scale_rows(CH // 2, r0)
        pltpu.sync_copy(
            xrowb.at[pl.ds(0, CH // 2)],
            xs_hbm.at[pl.ds(rel * NPAD + arow0 + r0, CH // 2)])

    plsc.subcore_barrier()

    pltpu.async_copy(xs_hbm.at[idx_src.at[0]], rows.at[0], sem)

    def edge_blk(j, _):
        buf = lax.rem(j, 2)
        @pl.when(j >= 1)
        def _free():
            pltpu.make_async_copy(
                rows.at[1 - buf], acc_sh.at[idx_dst.at[j - 1]], ssem).wait()
        @pl.when(j < NB - 1)
        def _issue():
            pltpu.async_copy(
                xs_hbm.at[idx_src.at[j + 1]], rows.at[1 - buf], sem)
        pltpu.make_async_copy(
            xs_hbm.at[idx_src.at[j]], rows.at[buf], sem).wait()
        pltpu.async_copy(rows.at[buf], acc_sh.at[idx_dst.at[j]], ssem,
                         add=True)
        return _
    lax.fori_loop(0, NB, edge_blk, None)
    pltpu.make_async_copy(
        rows.at[(NB - 1) % 2], acc_sh.at[idx_dst.at[NB - 1]], ssem).wait()

    plsc.subcore_barrier()

    @pl.when(rel == 0)
    def _dump0():
        pltpu.sync_copy(acc_sh.at[pl.ds(arow0, RPT)],
                        p0_hbm.at[pl.ds(arow0, RPT)])

    @pl.when(rel == 1)
    def _dump1():
        pltpu.sync_copy(acc_sh.at[pl.ds(arow0, RPT)],
                        p1_hbm.at[pl.ds(arow0, RPT)])


def _tc_mm_body(p0, p1, nd0, nd1, w0, w1, b0, b1, a, o):
    q0 = p0[...].astype(jnp.float32) * nd0[...]
    q1 = p1[...].astype(jnp.float32) * nd1[...]
    h = jnp.dot(q0, w0[...], preferred_element_type=jnp.float32,
                precision=lax.Precision.HIGHEST)
    h = h + jnp.dot(q1, w1[...], preferred_element_type=jnp.float32,
                    precision=lax.Precision.HIGHEST)
    h = h + b0[...] + b1[...]
    o[...] = jnp.where(h > 0, h, a[0, 0] * h)


def _pad_edges(src, dst, rel):
    src = src.reshape(NS, EC)
    dst = dst.reshape(NS, EC)
    npad = ECP - EC
    k = jnp.arange(npad, dtype=jnp.int32)[None, :] + 17 * jnp.arange(
        NS, dtype=jnp.int32)[:, None]
    padv = N + ((k * 7) % (NPAD - N))
    src = jnp.concatenate([src, padv], axis=1)
    dst = jnp.concatenate([dst, padv], axis=1)
    src = src + rel * NPAD
    return ((src << 14) | dst).reshape(NS, NB, BLK)


@jax.jit
def kernel(x, edge_index_rel0, edge_index_rel1, W0, b0, W1, b1, prelu_a):
    pk0 = _pad_edges(edge_index_rel0[0], edge_index_rel0[1], 0)
    pk1 = _pad_edges(edge_index_rel1[0], edge_index_rel1[1], 1)
    pk_all = jnp.stack([pk0, pk1])

    mesh = plsc.VectorSubcoreMesh(core_axis_name="c", subcore_axis_name="s")
    sc = pl.kernel(
        _sc_body,
        out_type=(
            jax.ShapeDtypeStruct((NC * NPAD, D), jnp.bfloat16),
            jax.ShapeDtypeStruct((NC * NPAD,), jnp.float32),
            jax.ShapeDtypeStruct((NPAD, D), jnp.bfloat16),
            jax.ShapeDtypeStruct((NPAD, D), jnp.bfloat16),
        ),
        mesh=mesh,
        compiler_params=pltpu.CompilerParams(
            needs_layout_passes=False, use_tc_tiling_on_sc=False),
        scratch_types=[
            pltpu.VMEM((NB, BLK), jnp.int32),
            pltpu.VMEM((NB, BLK), jnp.int32),
            pltpu.VMEM((NB, BLK), jnp.int32),
            pltpu.VMEM((2, BLK, D), jnp.bfloat16),
            pltpu.VMEM((CH, D), jnp.float32),
            pltpu.VMEM((CH, D), jnp.bfloat16),
            pltpu.VMEM((RPT,), jnp.float32),
            pltpu.VMEM((RPT,), jnp.float32),
            pltpu.VMEM((128,), jnp.float32),
            pltpu.VMEM((8, D), jnp.bfloat16),
            pltpu.VMEM((BLK,), jnp.float32),
            pltpu.VMEM_SHARED((NPAD, D), jnp.bfloat16),
            pltpu.VMEM_SHARED((NC * NPAD,), jnp.float32),
            pltpu.VMEM_SHARED((NPAD,), jnp.float32),
            pltpu.SemaphoreType.DMA,
            pltpu.SemaphoreType.DMA,
        ],
    )
    _, nd, p0, p1 = sc(x, pk_all)

    w0p = W0[PERM, :]
    w1p = W1[PERM, :]
    nd2 = nd.reshape(NC * NPAD, 1)
    nd0 = nd2[:NPAD]
    nd1 = nd2[NPAD:]
    bs = 400
    h = pl.pallas_call(
        _tc_mm_body,
        grid=(N // bs,),
        in_specs=[
            pl.BlockSpec((bs, D), lambda j: (j, 0)),
            pl.BlockSpec((bs, D), lambda j: (j, 0)),
            pl.BlockSpec((bs, 1), lambda j: (j, 0)),
            pl.BlockSpec((bs, 1), lambda j: (j, 0)),
            pl.BlockSpec((D, D), lambda j: (0, 0)),
            pl.BlockSpec((D, D), lambda j: (0, 0)),
            pl.BlockSpec((1, D), lambda j: (0, 0)),
            pl.BlockSpec((1, D), lambda j: (0, 0)),
            pl.BlockSpec((1, 1), lambda j: (0, 0)),
        ],
        out_specs=pl.BlockSpec((bs, D), lambda j: (j, 0)),
        out_shape=jax.ShapeDtypeStruct((N, D), jnp.float32),
    )(p0, p1, nd0, nd1, w0p, w1p, b0.reshape(1, D), b1.reshape(1, D),
      prelu_a.reshape(1, 1))
    return h


# speedup vs baseline: 1.0760x; 1.0760x over previous
import numpy as np

import jax
import jax.numpy as jnp
from jax import lax
from jax.experimental import pallas as pl
from jax.experimental.pallas import tpu as pltpu
from jax.experimental.pallas import tpu_sc as plsc

N = 10000
E = 320000
D = 128
NS = 16
NC = 2
NPAD = 10240
RPT = NPAD // NS
EC = E // NS
BLK = 128
NB = (EC + BLK - 1) // BLK
ECP = NB * BLK
CH = 32
NCH = RPT // CH

_g = np.arange(16)
_p32 = np.empty(32, np.int32)
_p32[0::2] = _g
_p32[1::2] = _g + 16
PERM = (np.arange(0, D, 32)[:, None] + _p32[None, :]).ravel()


def _rsqrt16(d):
    xi = plsc.bitcast(d, jnp.int32)
    yi = jnp.int32(0x5F3759DF) - (xi >> 1)
    y = plsc.bitcast(yi, jnp.float32)
    for _ in range(3):
        y = y * (1.5 - 0.5 * d * y * y)
    return y


def _sc_body(x_hbm, pk_hbm, xs_hbm, nd_hbm, p0_hbm, p1_hbm,
             pk, idx_src, idx_dst, rows, xrowf, xrowb, nsrc, ndbuf,
             z128, zrow, ones, acc_sh, degs_sh, degd_sh, sem, ssem):
    rel = lax.axis_index("c")
    tile = lax.axis_index("s")
    arow0 = tile * RPT

    z16 = jnp.zeros((16,), jnp.float32)
    zb = jnp.zeros((32,), jnp.bfloat16)
    for i in range(8):
        z128[pl.ds(i * 16, 16)] = z16
        ones[pl.ds(i * 16, 16)] = z16 + 1.0
        for k in range(D // 32):
            zrow[i, pl.ds(k * 32, 32)] = zb

    def zero_deg(i, _):
        pltpu.sync_copy(z128, degs_sh.at[pl.ds(rel * NPAD + arow0 + 128 * i, 128)])
        pltpu.sync_copy(z128, degd_sh.at[pl.ds(arow0 + 128 * i, 128)])
        return _
    lax.fori_loop(0, RPT // 128, zero_deg, None)

    def zero_acc(i, _):
        pltpu.sync_copy(zrow, acc_sh.at[pl.ds(arow0 + 8 * i, 8)])
        return _
    lax.fori_loop(0, RPT // 8, zero_acc, None)

    pltpu.sync_copy(pk_hbm.at[rel, tile], pk)

    def unpack_blk(j, _):
        for k in range(BLK // 16):
            s = pl.ds(k * 16, 16)
            v = pk[j, s]
            idx_src[j, s] = v >> 14
            idx_dst[j, s] = v & 16383
        return _
    lax.fori_loop(0, NB, unpack_blk, None)

    plsc.subcore_barrier()

    DW = 8

    def deg_blk(j, _):
        pltpu.async_copy(ones, degs_sh.at[idx_src.at[j]], ssem, add=True)
        pltpu.async_copy(ones, degd_sh.at[idx_dst.at[j]], sem, add=True)
        @pl.when(j >= DW)
        def _drain():
            pltpu.make_async_copy(
                ones, degs_sh.at[idx_src.at[j - DW]], ssem).wait()
            pltpu.make_async_copy(
                ones, degd_sh.at[idx_dst.at[j - DW]], sem).wait()
        return _
    lax.fori_loop(0, NB, deg_blk, None)

    def deg_drain(j, _):
        pltpu.make_async_copy(
            ones, degs_sh.at[idx_src.at[NB - DW + j]], ssem).wait()
        pltpu.make_async_copy(
            ones, degd_sh.at[idx_dst.at[NB - DW + j]], sem).wait()
        return _
    lax.fori_loop(0, DW, deg_drain, None)

    plsc.subcore_barrier()

    pltpu.sync_copy(degs_sh.at[pl.ds(rel * NPAD + arow0, RPT)], nsrc)
    pltpu.sync_copy(degd_sh.at[pl.ds(arow0, RPT)], ndbuf)

    def norm_blk(i, _):
        s = pl.ds(i * 16, 16)
        nsrc[s] = _rsqrt16(jnp.maximum(nsrc[s], 1.0))
        ndbuf[s] = _rsqrt16(jnp.maximum(ndbuf[s], 1.0))
        return _
    lax.fori_loop(0, RPT // 16, norm_blk, None)
    pltpu.sync_copy(ndbuf, nd_hbm.at[pl.ds(rel * NPAD + arow0, RPT)])

    def scale_rows(nrows, r0):
        for i in range(nrows):
            w = plsc.load_gather(nsrc, [jnp.full((16,), r0 + i, jnp.int32)])
            for k in range(D // 32):
                a = xrowf[i, pl.ds(k * 32, 16)] * w
                b = xrowf[i, pl.ds(k * 32 + 16, 16)] * w
                xrowb[i, pl.ds(k * 32, 32)] = plsc.pack(
                    a, b, format=plsc.PackFormat.INTERLEAVED)

    lim = jnp.clip(N - arow0, 0, RPT)
    nfull = lim // CH

    def stage_chunk(c, _):
        r0 = c * CH
        pltpu.sync_copy(x_hbm.at[pl.ds(arow0 + r0, CH)], xrowf)
        scale_rows(CH, r0)
        pltpu.sync_copy(
            xrowb, xs_hbm.at[pl.ds(rel * NPAD + arow0 + r0, CH)])
        return _
    lax.fori_loop(0, nfull, stage_chunk, None)

    @pl.when(nfull * CH < lim)
    def _tail():
        r0 = nfull * CH
        pltpu.sync_copy(x_hbm.at[pl.ds(arow0 + r0, CH // 2)],
                        xrowf.at[pl.ds(0, CH // 2)])
        scale_rows(CH // 2, r0)
        pltpu.sync_copy(
            xrowb.at[pl.ds(0, CH // 2)],
            xs_hbm.at[pl.ds(rel * NPAD + arow0 + r0, CH // 2)])

    plsc.subcore_barrier()

    pltpu.async_copy(xs_hbm.at[idx_src.at[0]], rows.at[0], sem)

    def edge_blk(j, _):
        buf = lax.rem(j, 2)
        @pl.when(j >= 1)
        def _free():
            pltpu.make_async_copy(
                rows.at[1 - buf], acc_sh.at[idx_dst.at[j - 1]], ssem).wait()
        @pl.when(j < NB - 1)
        def _issue():
            pltpu.async_copy(
                xs_hbm.at[idx_src.at[j + 1]], rows.at[1 - buf], sem)
        pltpu.make_async_copy(
            xs_hbm.at[idx_src.at[j]], rows.at[buf], sem).wait()
        pltpu.async_copy(rows.at[buf], acc_sh.at[idx_dst.at[j]], ssem,
                         add=True)
        return _
    lax.fori_loop(0, NB, edge_blk, None)
    pltpu.make_async_copy(
        rows.at[(NB - 1) % 2], acc_sh.at[idx_dst.at[NB - 1]], ssem).wait()

    plsc.subcore_barrier()

    @pl.when(rel == 0)
    def _dump0():
        pltpu.sync_copy(acc_sh.at[pl.ds(arow0, RPT)],
                        p0_hbm.at[pl.ds(arow0, RPT)])

    @pl.when(rel == 1)
    def _dump1():
        pltpu.sync_copy(acc_sh.at[pl.ds(arow0, RPT)],
                        p1_hbm.at[pl.ds(arow0, RPT)])


def _tc_mm_body(p0, p1, nd0, nd1, w0, w1, b0, b1, a, o):
    q0 = p0[...].astype(jnp.float32) * nd0[...]
    q1 = p1[...].astype(jnp.float32) * nd1[...]
    h = jnp.dot(q0, w0[...], preferred_element_type=jnp.float32,
                precision=lax.Precision.HIGHEST)
    h = h + jnp.dot(q1, w1[...], preferred_element_type=jnp.float32,
                    precision=lax.Precision.HIGHEST)
    h = h + b0[...] + b1[...]
    o[...] = jnp.where(h > 0, h, a[0, 0] * h)


def _pad_edges(src, dst, rel):
    src = src.reshape(NS, EC)
    dst = dst.reshape(NS, EC)
    npad = ECP - EC
    k = jnp.arange(npad, dtype=jnp.int32)[None, :] + 17 * jnp.arange(
        NS, dtype=jnp.int32)[:, None]
    padv = N + ((k * 7) % (NPAD - N))
    src = jnp.concatenate([src, padv], axis=1)
    dst = jnp.concatenate([dst, padv], axis=1)
    src = src + rel * NPAD
    return ((src << 14) | dst).reshape(NS, NB, BLK)


@jax.jit
def kernel(x, edge_index_rel0, edge_index_rel1, W0, b0, W1, b1, prelu_a):
    pk0 = _pad_edges(edge_index_rel0[0], edge_index_rel0[1], 0)
    pk1 = _pad_edges(edge_index_rel1[0], edge_index_rel1[1], 1)
    pk_all = jnp.stack([pk0, pk1])

    mesh = plsc.VectorSubcoreMesh(core_axis_name="c", subcore_axis_name="s")
    sc = pl.kernel(
        _sc_body,
        out_type=(
            jax.ShapeDtypeStruct((NC * NPAD, D), jnp.bfloat16),
            jax.ShapeDtypeStruct((NC * NPAD,), jnp.float32),
            jax.ShapeDtypeStruct((NPAD, D), jnp.bfloat16),
            jax.ShapeDtypeStruct((NPAD, D), jnp.bfloat16),
        ),
        mesh=mesh,
        compiler_params=pltpu.CompilerParams(
            needs_layout_passes=False, use_tc_tiling_on_sc=False),
        scratch_types=[
            pltpu.VMEM((NB, BLK), jnp.int32),
            pltpu.VMEM((NB, BLK), jnp.int32),
            pltpu.VMEM((NB, BLK), jnp.int32),
            pltpu.VMEM((2, BLK, D), jnp.bfloat16),
            pltpu.VMEM((CH, D), jnp.float32),
            pltpu.VMEM((CH, D), jnp.bfloat16),
            pltpu.VMEM((RPT,), jnp.float32),
            pltpu.VMEM((RPT,), jnp.float32),
            pltpu.VMEM((128,), jnp.float32),
            pltpu.VMEM((8, D), jnp.bfloat16),
            pltpu.VMEM((BLK,), jnp.float32),
            pltpu.VMEM_SHARED((NPAD, D), jnp.bfloat16),
            pltpu.VMEM_SHARED((NC * NPAD,), jnp.float32),
            pltpu.VMEM_SHARED((NPAD,), jnp.float32),
            pltpu.SemaphoreType.DMA,
            pltpu.SemaphoreType.DMA,
        ],
    )
    _, nd, p0, p1 = sc(x, pk_all)

    w0p = W0[PERM, :]
    w1p = W1[PERM, :]
    nd2 = nd.reshape(NC * NPAD, 1)
    nd0 = nd2[:NPAD]
    nd1 = nd2[NPAD:]
    bs = 400
    h = pl.pallas_call(
        _tc_mm_body,
        grid=(N // bs,),
        in_specs=[
            pl.BlockSpec((bs, D), lambda j: (j, 0)),
            pl.BlockSpec((bs, D), lambda j: (j, 0)),
            pl.BlockSpec((bs, 1), lambda j: (j, 0)),
            pl.BlockSpec((bs, 1), lambda j: (j, 0)),
            pl.BlockSpec((D, D), lambda j: (0, 0)),
            pl.BlockSpec((D, D), lambda j: (0, 0)),
            pl.BlockSpec((1, D), lambda j: (0, 0)),
            pl.BlockSpec((1, D), lambda j: (0, 0)),
            pl.BlockSpec((1, 1), lambda j: (0, 0)),
        ],
        out_specs=pl.BlockSpec((bs, D), lambda j: (j, 0)),
        out_shape=jax.ShapeDtypeStruct((N, D), jnp.float32),
    )(p0, p1, nd0, nd1, w0p, w1p, b0.reshape(1, D), b1.reshape(1, D),
      prelu_a.reshape(1, 1))
    return h


# speedup vs baseline: 1.2208x; 1.1346x over previous
import numpy as np

import jax
import jax.numpy as jnp
from jax import lax
from jax.experimental import pallas as pl
from jax.experimental.pallas import tpu as pltpu
from jax.experimental.pallas import tpu_sc as plsc

N = 10000
E = 320000
D = 128
NS = 16
NC = 2
NPAD = 10240
RPT = NPAD // NS
EC = E // NS
BLK = 128
NB = (EC + BLK - 1) // BLK
ECP = NB * BLK
CH = 32
NCH = RPT // CH

_g = np.arange(16)
_p32 = np.empty(32, np.int32)
_p32[0::2] = _g
_p32[1::2] = _g + 16
PERM = (np.arange(0, D, 32)[:, None] + _p32[None, :]).ravel()


def _rsqrt16(d):
    xi = plsc.bitcast(d, jnp.int32)
    yi = jnp.int32(0x5F3759DF) - (xi >> 1)
    y = plsc.bitcast(yi, jnp.float32)
    for _ in range(3):
        y = y * (1.5 - 0.5 * d * y * y)
    return y


def _sc_body(x_hbm, pk_hbm, xs_hbm, nd_hbm, p0_hbm, p1_hbm,
             idx_src, idx_dst, rows, xrowf, xrowb, nsrc, ndbuf,
             z128, zrow, ones, acc_sh, degs_sh, degd_sh, sem, ssem):
    rel = lax.axis_index("c")
    tile = lax.axis_index("s")
    arow0 = tile * RPT

    z16 = jnp.zeros((16,), jnp.float32)
    zb = jnp.zeros((32,), jnp.bfloat16)
    for i in range(8):
        z128[pl.ds(i * 16, 16)] = z16
        ones[pl.ds(i * 16, 16)] = z16 + 1.0
        for k in range(D // 32):
            zrow[i, pl.ds(k * 32, 32)] = zb

    def zero_deg(i, _):
        pltpu.sync_copy(z128, degs_sh.at[pl.ds(rel * NPAD + arow0 + 128 * i, 128)])
        pltpu.sync_copy(z128, degd_sh.at[pl.ds(arow0 + 128 * i, 128)])
        return _
    lax.fori_loop(0, RPT // 128, zero_deg, None)

    def zero_acc(i, _):
        pltpu.sync_copy(zrow, acc_sh.at[pl.ds(arow0 + 8 * i, 8)])
        return _
    lax.fori_loop(0, RPT // 8, zero_acc, None)

    pltpu.sync_copy(pk_hbm.at[rel, tile], idx_src)

    def unpack_blk(j, _):
        for k in range(BLK // 16):
            s = pl.ds(k * 16, 16)
            v = idx_src[j, s]
            idx_dst[j, s] = v & 16383
            idx_src[j, s] = v >> 14
        return _
    lax.fori_loop(0, NB, unpack_blk, None)

    plsc.subcore_barrier()

    DW = 8

    def deg_blk(j, _):
        pltpu.async_copy(ones, degs_sh.at[idx_src.at[j]], ssem, add=True)
        pltpu.async_copy(ones, degd_sh.at[idx_dst.at[j]], sem, add=True)
        @pl.when(j >= DW)
        def _drain():
            pltpu.make_async_copy(
                ones, degs_sh.at[idx_src.at[j - DW]], ssem).wait()
            pltpu.make_async_copy(
                ones, degd_sh.at[idx_dst.at[j - DW]], sem).wait()
        return _
    lax.fori_loop(0, NB, deg_blk, None)

    def deg_drain(j, _):
        pltpu.make_async_copy(
            ones, degs_sh.at[idx_src.at[NB - DW + j]], ssem).wait()
        pltpu.make_async_copy(
            ones, degd_sh.at[idx_dst.at[NB - DW + j]], sem).wait()
        return _
    lax.fori_loop(0, DW, deg_drain, None)

    plsc.subcore_barrier()

    pltpu.sync_copy(degs_sh.at[pl.ds(rel * NPAD + arow0, RPT)], nsrc)
    pltpu.sync_copy(degd_sh.at[pl.ds(arow0, RPT)], ndbuf)

    def norm_blk(i, _):
        s = pl.ds(i * 16, 16)
        nsrc[s] = _rsqrt16(jnp.maximum(nsrc[s], 1.0))
        ndbuf[s] = _rsqrt16(jnp.maximum(ndbuf[s], 1.0))
        return _
    lax.fori_loop(0, RPT // 16, norm_blk, None)
    pltpu.sync_copy(ndbuf, nd_hbm.at[pl.ds(rel * NPAD + arow0, RPT)])

    def scale_rows(nrows, r0):
        for i in range(nrows):
            w = plsc.load_gather(nsrc, [jnp.full((16,), r0 + i, jnp.int32)])
            for k in range(D // 32):
                a = xrowf[i, pl.ds(k * 32, 16)] * w
                b = xrowf[i, pl.ds(k * 32 + 16, 16)] * w
                xrowb[i, pl.ds(k * 32, 32)] = plsc.pack(
                    a, b, format=plsc.PackFormat.INTERLEAVED)

    lim = jnp.clip(N - arow0, 0, RPT)
    nfull = lim // CH

    def stage_chunk(c, _):
        r0 = c * CH
        pltpu.sync_copy(x_hbm.at[pl.ds(arow0 + r0, CH)], xrowf)
        scale_rows(CH, r0)
        pltpu.sync_copy(
            xrowb, xs_hbm.at[pl.ds(rel * NPAD + arow0 + r0, CH)])
        return _
    lax.fori_loop(0, nfull, stage_chunk, None)

    @pl.when(nfull * CH < lim)
    def _tail():
        r0 = nfull * CH
        pltpu.sync_copy(x_hbm.at[pl.ds(arow0 + r0, CH // 2)],
                        xrowf.at[pl.ds(0, CH // 2)])
        scale_rows(CH // 2, r0)
        pltpu.sync_copy(
            xrowb.at[pl.ds(0, CH // 2)],
            xs_hbm.at[pl.ds(rel * NPAD + arow0 + r0, CH // 2)])

    plsc.subcore_barrier()

    pltpu.async_copy(xs_hbm.at[idx_src.at[0]], rows.at[0], sem)
    pltpu.async_copy(xs_hbm.at[idx_src.at[1]], rows.at[1], sem)

    def edge_blk(j, _):
        buf = lax.rem(j, 3)
        @pl.when(j >= 2)
        def _free():
            pltpu.make_async_copy(
                rows.at[lax.rem(j + 1, 3)], acc_sh.at[idx_dst.at[j - 2]],
                ssem).wait()
        @pl.when(j < NB - 2)
        def _issue():
            pltpu.async_copy(
                xs_hbm.at[idx_src.at[j + 2]], rows.at[lax.rem(j + 2, 3)], sem)
        pltpu.make_async_copy(
            xs_hbm.at[idx_src.at[j]], rows.at[buf], sem).wait()
        pltpu.async_copy(rows.at[buf], acc_sh.at[idx_dst.at[j]], ssem,
                         add=True)
        return _
    lax.fori_loop(0, NB, edge_blk, None)
    pltpu.make_async_copy(
        rows.at[(NB - 2) % 3], acc_sh.at[idx_dst.at[NB - 2]], ssem).wait()
    pltpu.make_async_copy(
        rows.at[(NB - 1) % 3], acc_sh.at[idx_dst.at[NB - 1]], ssem).wait()

    plsc.subcore_barrier()

    @pl.when(rel == 0)
    def _dump0():
        pltpu.sync_copy(acc_sh.at[pl.ds(arow0, RPT)],
                        p0_hbm.at[pl.ds(arow0, RPT)])

    @pl.when(rel == 1)
    def _dump1():
        pltpu.sync_copy(acc_sh.at[pl.ds(arow0, RPT)],
                        p1_hbm.at[pl.ds(arow0, RPT)])


def _tc_mm_body(p0, p1, nd0, nd1, w0, w1, b0, b1, a, o):
    q0 = p0[...].astype(jnp.float32) * nd0[...]
    q1 = p1[...].astype(jnp.float32) * nd1[...]
    h = jnp.dot(q0, w0[...], preferred_element_type=jnp.float32,
                precision=lax.Precision.HIGHEST)
    h = h + jnp.dot(q1, w1[...], preferred_element_type=jnp.float32,
                    precision=lax.Precision.HIGHEST)
    h = h + b0[...] + b1[...]
    o[...] = jnp.where(h > 0, h, a[0, 0] * h)


def _pad_edges(src, dst, rel):
    src = src.reshape(NS, EC)
    dst = dst.reshape(NS, EC)
    npad = ECP - EC
    k = jnp.arange(npad, dtype=jnp.int32)[None, :] + 17 * jnp.arange(
        NS, dtype=jnp.int32)[:, None]
    padv = N + ((k * 7) % (NPAD - N))
    src = jnp.concatenate([src, padv], axis=1)
    dst = jnp.concatenate([dst, padv], axis=1)
    src = src + rel * NPAD
    return ((src << 14) | dst).reshape(NS, NB, BLK)


@jax.jit
def kernel(x, edge_index_rel0, edge_index_rel1, W0, b0, W1, b1, prelu_a):
    pk0 = _pad_edges(edge_index_rel0[0], edge_index_rel0[1], 0)
    pk1 = _pad_edges(edge_index_rel1[0], edge_index_rel1[1], 1)
    pk_all = jnp.stack([pk0, pk1])

    mesh = plsc.VectorSubcoreMesh(core_axis_name="c", subcore_axis_name="s")
    sc = pl.kernel(
        _sc_body,
        out_type=(
            jax.ShapeDtypeStruct((NC * NPAD, D), jnp.bfloat16),
            jax.ShapeDtypeStruct((NC * NPAD,), jnp.float32),
            jax.ShapeDtypeStruct((NPAD, D), jnp.bfloat16),
            jax.ShapeDtypeStruct((NPAD, D), jnp.bfloat16),
        ),
        mesh=mesh,
        compiler_params=pltpu.CompilerParams(
            needs_layout_passes=False, use_tc_tiling_on_sc=False),
        scratch_types=[
            pltpu.VMEM((NB, BLK), jnp.int32),
            pltpu.VMEM((NB, BLK), jnp.int32),
            pltpu.VMEM((3, BLK, D), jnp.bfloat16),
            pltpu.VMEM((CH, D), jnp.float32),
            pltpu.VMEM((CH, D), jnp.bfloat16),
            pltpu.VMEM((RPT,), jnp.float32),
            pltpu.VMEM((RPT,), jnp.float32),
            pltpu.VMEM((128,), jnp.float32),
            pltpu.VMEM((8, D), jnp.bfloat16),
            pltpu.VMEM((BLK,), jnp.float32),
            pltpu.VMEM_SHARED((NPAD, D), jnp.bfloat16),
            pltpu.VMEM_SHARED((NC * NPAD,), jnp.float32),
            pltpu.VMEM_SHARED((NPAD,), jnp.float32),
            pltpu.SemaphoreType.DMA,
            pltpu.SemaphoreType.DMA,
        ],
    )
    _, nd, p0, p1 = sc(x, pk_all)

    w0p = W0[PERM, :]
    w1p = W1[PERM, :]
    nd2 = nd.reshape(NC * NPAD, 1)
    nd0 = nd2[:NPAD]
    nd1 = nd2[NPAD:]
    bs = 400
    h = pl.pallas_call(
        _tc_mm_body,
        grid=(N // bs,),
        in_specs=[
            pl.BlockSpec((bs, D), lambda j: (j, 0)),
            pl.BlockSpec((bs, D), lambda j: (j, 0)),
            pl.BlockSpec((bs, 1), lambda j: (j, 0)),
            pl.BlockSpec((bs, 1), lambda j: (j, 0)),
            pl.BlockSpec((D, D), lambda j: (0, 0)),
            pl.BlockSpec((D, D), lambda j: (0, 0)),
            pl.BlockSpec((1, D), lambda j: (0, 0)),
            pl.BlockSpec((1, D), lambda j: (0, 0)),
            pl.BlockSpec((1, 1), lambda j: (0, 0)),
        ],
        out_specs=pl.BlockSpec((bs, D), lambda j: (j, 0)),
        out_shape=jax.ShapeDtypeStruct((N, D), jnp.float32),
    )(p0, p1, nd0, nd1, w0p, w1p, b0.reshape(1, D), b1.reshape(1, D),
      prelu_a.reshape(1, 1))
    return h
